# speedup vs baseline: 1.2568x; 1.2568x over previous
import jax
import jax.numpy as jnp
from jax import lax
from jax.experimental import pallas as pl
from jax.experimental.pallas import tpu as pltpu
from jax.experimental.pallas import tpu_sc as plsc

BATCH = 16384
NC = 2
NS = 16
NW = NC * NS
LANES = 16
ROW = 128
ROWS_PER_W = BATCH // (NW * ROW)


def _softplus(x):
    u = jnp.exp(-jnp.abs(x))
    s = u / (2.0 + u)
    s2 = s * s
    log1p_u = 2.0 * s * (1.0 + s2 * (1.0 / 3.0 + s2 * (1.0 / 5.0 + s2 * (1.0 / 7.0 + s2 * (1.0 / 9.0)))))
    return jnp.maximum(x, 0.0) + log1p_u


def _body(aidx_hbm, pidx_hbm, loss_hbm, tau_a_hbm, tau_p_hbm, out_hbm,
          aidx_v, pidx_v, loss_v, va_v, vp_v, out_v, sem):
    wid = lax.axis_index("s") * NC + lax.axis_index("c")
    r0 = wid * ROWS_PER_W

    cp_a = pltpu.async_copy(aidx_hbm.at[pl.ds(r0, ROWS_PER_W)], aidx_v, sem)
    cp_p = pltpu.async_copy(pidx_hbm.at[pl.ds(r0, ROWS_PER_W)], pidx_v, sem)
    cp_l = pltpu.async_copy(loss_hbm.at[pl.ds(r0, ROWS_PER_W)], loss_v, sem)
    cp_a.wait()
    cp_p.wait()

    gathers = []
    for j in range(ROWS_PER_W):
        gathers.append(pltpu.async_copy(tau_a_hbm.at[aidx_v.at[j]], va_v.at[j], sem))
        gathers.append(pltpu.async_copy(tau_p_hbm.at[pidx_v.at[j]], vp_v.at[j], sem))
    cp_l.wait()
    for g in gathers:
        g.wait()

    for j in range(ROWS_PER_W):
        for i in range(ROW // LANES):
            sl = pl.ds(i * LANES, LANES)
            lam = _softplus(va_v[j, sl]) * _softplus(vp_v[j, sl])
            out_v[j, sl] = (loss_v[j, sl] - 1.0) * lam

    pltpu.sync_copy(out_v, out_hbm.at[pl.ds(r0, ROWS_PER_W)])


@jax.jit
def kernel(amplitude_idxs, phase_idxs, losses, tau_amplitude, tau_phase):
    n_rows = BATCH // ROW
    mesh = plsc.VectorSubcoreMesh(core_axis_name="c", subcore_axis_name="s")
    run = pl.kernel(
        _body,
        out_type=jax.ShapeDtypeStruct((n_rows, ROW), jnp.float32),
        mesh=mesh,
        scratch_types=[
            pltpu.VMEM((ROWS_PER_W, ROW), jnp.int32),
            pltpu.VMEM((ROWS_PER_W, ROW), jnp.int32),
            pltpu.VMEM((ROWS_PER_W, ROW), jnp.float32),
            pltpu.VMEM((ROWS_PER_W, ROW), jnp.float32),
            pltpu.VMEM((ROWS_PER_W, ROW), jnp.float32),
            pltpu.VMEM((ROWS_PER_W, ROW), jnp.float32),
            pltpu.SemaphoreType.DMA,
        ],
    )
    out = run(
        amplitude_idxs.reshape(n_rows, ROW),
        phase_idxs.reshape(n_rows, ROW),
        losses.reshape(n_rows, ROW),
        tau_amplitude,
        tau_phase,
    )
    return out.reshape(BATCH)


# speedup vs baseline: 1.2728x; 1.0128x over previous
import jax
import jax.numpy as jnp
from jax import lax
from jax.experimental import pallas as pl
from jax.experimental.pallas import tpu as pltpu
from jax.experimental.pallas import tpu_sc as plsc

BATCH = 16384
NC = 2
NS = 16
NW = NC * NS
LANES = 16
ROW = 128
ROWS_PER_W = BATCH // (NW * ROW)


def _softplus(x):
    u = jnp.exp(-jnp.abs(x))
    s = u / (2.0 + u)
    s2 = s * s
    log1p_u = 2.0 * s * (1.0 + s2 * (1.0 / 3.0 + s2 * (1.0 / 5.0)))
    return jnp.maximum(x, 0.0) + log1p_u


def _body(aidx_hbm, pidx_hbm, loss_hbm, tau_a_hbm, tau_p_hbm, out_hbm,
          aidx_v, pidx_v, loss_v, va_v, vp_v, out_v,
          sem_a, sem_p, sem_l, sem_o, *row_sems):
    wid = lax.axis_index("s") * NC + lax.axis_index("c")
    r0 = wid * ROWS_PER_W

    cp_a = pltpu.async_copy(aidx_hbm.at[pl.ds(r0, ROWS_PER_W)], aidx_v, sem_a)
    cp_p = pltpu.async_copy(pidx_hbm.at[pl.ds(r0, ROWS_PER_W)], pidx_v, sem_p)
    cp_l = pltpu.async_copy(loss_hbm.at[pl.ds(r0, ROWS_PER_W)], loss_v, sem_l)

    cp_a.wait()
    ga = [pltpu.async_copy(tau_a_hbm.at[aidx_v.at[j]], va_v.at[j], row_sems[j])
          for j in range(ROWS_PER_W)]
    cp_p.wait()
    gp = [pltpu.async_copy(tau_p_hbm.at[pidx_v.at[j]], vp_v.at[j], row_sems[j])
          for j in range(ROWS_PER_W)]
    cp_l.wait()

    outs = []
    for j in range(ROWS_PER_W):
        ga[j].wait()
        gp[j].wait()
        for i in range(ROW // LANES):
            sl = pl.ds(i * LANES, LANES)
            lam = _softplus(va_v[j, sl]) * _softplus(vp_v[j, sl])
            out_v[j, sl] = (loss_v[j, sl] - 1.0) * lam
        outs.append(pltpu.async_copy(out_v.at[j], out_hbm.at[r0 + j], sem_o))
    for o in outs:
        o.wait()


@jax.jit
def kernel(amplitude_idxs, phase_idxs, losses, tau_amplitude, tau_phase):
    n_rows = BATCH // ROW
    mesh = plsc.VectorSubcoreMesh(core_axis_name="c", subcore_axis_name="s")
    run = pl.kernel(
        _body,
        out_type=jax.ShapeDtypeStruct((n_rows, ROW), jnp.float32),
        mesh=mesh,
        scratch_types=[
            pltpu.VMEM((ROWS_PER_W, ROW), jnp.int32),
            pltpu.VMEM((ROWS_PER_W, ROW), jnp.int32),
            pltpu.VMEM((ROWS_PER_W, ROW), jnp.float32),
            pltpu.VMEM((ROWS_PER_W, ROW), jnp.float32),
            pltpu.VMEM((ROWS_PER_W, ROW), jnp.float32),
            pltpu.VMEM((ROWS_PER_W, ROW), jnp.float32),
            pltpu.SemaphoreType.DMA,
            pltpu.SemaphoreType.DMA,
            pltpu.SemaphoreType.DMA,
            pltpu.SemaphoreType.DMA,
            pltpu.SemaphoreType.DMA,
            pltpu.SemaphoreType.DMA,
            pltpu.SemaphoreType.DMA,
            pltpu.SemaphoreType.DMA,
        ],
    )
    out = run(
        amplitude_idxs.reshape(n_rows, ROW),
        phase_idxs.reshape(n_rows, ROW),
        losses.reshape(n_rows, ROW),
        tau_amplitude,
        tau_phase,
    )
    return out.reshape(BATCH)


# speedup vs baseline: 1.3099x; 1.0291x over previous
import jax
import jax.numpy as jnp
from jax import lax
from jax.experimental import pallas as pl
from jax.experimental.pallas import tpu as pltpu
from jax.experimental.pallas import tpu_sc as plsc

BATCH = 16384
NC = 2
NS = 16
NW = NC * NS
LANES = 16
ROW = 128
ROWS_PER_W = BATCH // (NW * ROW)


_C = (0.9999818714624722, -0.4991878401334513, 0.3244117606313534,
      -0.2086695713434938, 0.1002871370282292, -0.023689236277343366)


def _softplus(x):
    u = jnp.exp(-jnp.abs(x))
    p = _C[5]
    for c in (_C[4], _C[3], _C[2], _C[1], _C[0]):
        p = p * u + c
    return jnp.maximum(x, 0.0) + u * p


def _body(aidx_hbm, pidx_hbm, loss_hbm, tau_a_hbm, tau_p_hbm, out_hbm,
          aidx_v, pidx_v, loss_v, va_v, vp_v, out_v,
          sem_a, sem_p, sem_l, sem_o, *row_sems):
    wid = lax.axis_index("s") * NC + lax.axis_index("c")
    r0 = wid * ROWS_PER_W

    cp_a = pltpu.async_copy(aidx_hbm.at[pl.ds(r0, ROWS_PER_W)], aidx_v, sem_a)
    cp_p = pltpu.async_copy(pidx_hbm.at[pl.ds(r0, ROWS_PER_W)], pidx_v, sem_p)
    cp_l = pltpu.async_copy(loss_hbm.at[pl.ds(r0, ROWS_PER_W)], loss_v, sem_l)

    cp_a.wait()
    cp_p.wait()
    ga, gp = [], []
    for j in range(ROWS_PER_W):
        ga.append(pltpu.async_copy(tau_a_hbm.at[aidx_v.at[j]], va_v.at[j], row_sems[j]))
        gp.append(pltpu.async_copy(tau_p_hbm.at[pidx_v.at[j]], vp_v.at[j], row_sems[j]))
    cp_l.wait()

    outs = []
    for j in range(ROWS_PER_W):
        ga[j].wait()
        gp[j].wait()
        for i in range(ROW // LANES):
            sl = pl.ds(i * LANES, LANES)
            lam = _softplus(va_v[j, sl]) * _softplus(vp_v[j, sl])
            out_v[j, sl] = (loss_v[j, sl] - 1.0) * lam
        outs.append(pltpu.async_copy(out_v.at[j], out_hbm.at[r0 + j], sem_o))
    for o in outs:
        o.wait()


@jax.jit
def kernel(amplitude_idxs, phase_idxs, losses, tau_amplitude, tau_phase):
    n_rows = BATCH // ROW
    mesh = plsc.VectorSubcoreMesh(core_axis_name="c", subcore_axis_name="s")
    run = pl.kernel(
        _body,
        out_type=jax.ShapeDtypeStruct((n_rows, ROW), jnp.float32),
        mesh=mesh,
        scratch_types=[
            pltpu.VMEM((ROWS_PER_W, ROW), jnp.int32),
            pltpu.VMEM((ROWS_PER_W, ROW), jnp.int32),
            pltpu.VMEM((ROWS_PER_W, ROW), jnp.float32),
            pltpu.VMEM((ROWS_PER_W, ROW), jnp.float32),
            pltpu.VMEM((ROWS_PER_W, ROW), jnp.float32),
            pltpu.VMEM((ROWS_PER_W, ROW), jnp.float32),
            pltpu.SemaphoreType.DMA,
            pltpu.SemaphoreType.DMA,
            pltpu.SemaphoreType.DMA,
            pltpu.SemaphoreType.DMA,
            pltpu.SemaphoreType.DMA,
            pltpu.SemaphoreType.DMA,
            pltpu.SemaphoreType.DMA,
            pltpu.SemaphoreType.DMA,
        ],
    )
    out = run(
        amplitude_idxs.reshape(n_rows, ROW),
        phase_idxs.reshape(n_rows, ROW),
        losses.reshape(n_rows, ROW),
        tau_amplitude,
        tau_phase,
    )
    return out.reshape(BATCH)


# speedup vs baseline: 1.3164x; 1.0050x over previous
import jax
import jax.numpy as jnp
from jax import lax
from jax.experimental import pallas as pl
from jax.experimental.pallas import tpu as pltpu
from jax.experimental.pallas import tpu_sc as plsc

BATCH = 16384
NC = 2
NS = 16
NW = NC * NS
LANES = 16
CHUNK = BATCH // NW

_C = (0.9999818714624722, -0.4991878401334513, 0.3244117606313534,
      -0.2086695713434938, 0.1002871370282292, -0.023689236277343366)


def _softplus(x):
    u = jnp.exp(-jnp.abs(x))
    p = _C[5]
    for c in (_C[4], _C[3], _C[2], _C[1], _C[0]):
        p = p * u + c
    return jnp.maximum(x, 0.0) + u * p


def _body(aidx_hbm, pidx_hbm, loss_hbm, tau_a_hbm, tau_p_hbm, out_hbm,
          aidx_v, pidx_v, loss_v, va_v, vp_v, out_v,
          sem_a, sem_p, sem_l, sem_ga, sem_gp):
    wid = lax.axis_index("s") * NC + lax.axis_index("c")
    base = wid * CHUNK

    cp_a = pltpu.async_copy(aidx_hbm.at[pl.ds(base, CHUNK)], aidx_v, sem_a)
    cp_p = pltpu.async_copy(pidx_hbm.at[pl.ds(base, CHUNK)], pidx_v, sem_p)
    cp_l = pltpu.async_copy(loss_hbm.at[pl.ds(base, CHUNK)], loss_v, sem_l)

    cp_a.wait()
    ga = pltpu.async_copy(tau_a_hbm.at[aidx_v], va_v, sem_ga)
    cp_p.wait()
    gp = pltpu.async_copy(tau_p_hbm.at[pidx_v], vp_v, sem_gp)
    cp_l.wait()

    for i in range(CHUNK // LANES):
        sl = pl.ds(i * LANES, LANES)
        loss_v[sl] = loss_v[sl] - 1.0

    ga.wait()
    gp.wait()
    for i in range(CHUNK // LANES):
        sl = pl.ds(i * LANES, LANES)
        out_v[sl] = loss_v[sl] * (_softplus(va_v[sl]) * _softplus(vp_v[sl]))

    pltpu.sync_copy(out_v, out_hbm.at[pl.ds(base, CHUNK)])


@jax.jit
def kernel(amplitude_idxs, phase_idxs, losses, tau_amplitude, tau_phase):
    mesh = plsc.VectorSubcoreMesh(core_axis_name="c", subcore_axis_name="s")
    run = pl.kernel(
        _body,
        out_type=jax.ShapeDtypeStruct((BATCH,), jnp.float32),
        mesh=mesh,
        scratch_types=[
            pltpu.VMEM((CHUNK,), jnp.int32),
            pltpu.VMEM((CHUNK,), jnp.int32),
            pltpu.VMEM((CHUNK,), jnp.float32),
            pltpu.VMEM((CHUNK,), jnp.float32),
            pltpu.VMEM((CHUNK,), jnp.float32),
            pltpu.VMEM((CHUNK,), jnp.float32),
            pltpu.SemaphoreType.DMA,
            pltpu.SemaphoreType.DMA,
            pltpu.SemaphoreType.DMA,
            pltpu.SemaphoreType.DMA,
            pltpu.SemaphoreType.DMA,
        ],
    )
    return run(amplitude_idxs, phase_idxs, losses, tau_amplitude, tau_phase)
